# baseline (speedup 1.0000x reference)
import functools

import jax
import jax.numpy as jnp
from jax import lax
from jax.experimental import pallas as pl
from jax.experimental.pallas import tpu as pltpu
from jax.experimental.pallas import tpu_sc as plsc

VOCAB = 1000000
EMB = 64
B = 4096
L = 200

N = B * L
NC, NS = 2, 16
NW = NC * NS
CHUNK = 128
GROUP = 8
ROWS = CHUNK * GROUP
PER_W = N // NW
STEPS = PER_W // ROWS

_MESH = plsc.VectorSubcoreMesh(core_axis_name="c", subcore_axis_name="s")


@functools.partial(
    pl.kernel,
    out_type=jax.ShapeDtypeStruct((N, EMB), jnp.float32),
    mesh=_MESH,
    compiler_params=pltpu.CompilerParams(use_tc_tiling_on_sc=False),
    scratch_types=[
        pltpu.VMEM((GROUP, CHUNK), jnp.int32),
        pltpu.VMEM((ROWS, EMB), jnp.float32),
        pltpu.SemaphoreType.DMA,
    ],
)
def _gather_kernel(src_hbm, table_hbm, out_hbm, idx_v, rows_v, sem):
    wid = lax.axis_index("s") * NC + lax.axis_index("c")
    row0 = wid * (PER_W // CHUNK)

    def step(g, carry):
        r = row0 + g * GROUP
        pltpu.sync_copy(src_hbm.at[pl.ds(r, GROUP)], idx_v)
        copies = []
        for j in range(GROUP):
            copies.append(pltpu.async_copy(
                table_hbm.at[idx_v.at[j]],
                rows_v.at[pl.ds(j * CHUNK, CHUNK)],
                sem,
            ))
        for c in copies:
            c.wait()
        pltpu.sync_copy(rows_v, out_hbm.at[pl.ds(r * CHUNK, ROWS)])
        return carry

    lax.fori_loop(0, STEPS, step, 0)


def kernel(src, mask, emb_table):
    del mask
    src2d = src.reshape(N // CHUNK, CHUNK)
    out = _gather_kernel(src2d, emb_table)
    return out.reshape(B, L, EMB)


# speedup vs baseline: 1.0179x; 1.0179x over previous
import functools

import jax
import jax.numpy as jnp
from jax import lax
from jax.experimental import pallas as pl
from jax.experimental.pallas import tpu as pltpu
from jax.experimental.pallas import tpu_sc as plsc

VOCAB = 1000000
EMB = 64
B = 4096
L = 200

N = B * L
NC, NS = 2, 16
NW = NC * NS
PER_W = N // NW
CHUNK = 320
STEPS = PER_W // CHUNK
NBUF = 4
OUTER = STEPS // NBUF

_MESH = plsc.VectorSubcoreMesh(core_axis_name="c", subcore_axis_name="s")


@functools.partial(
    pl.kernel,
    out_type=jax.ShapeDtypeStruct((N, EMB), jnp.float32),
    mesh=_MESH,
    compiler_params=pltpu.CompilerParams(use_tc_tiling_on_sc=False),
    scratch_types=[
        pltpu.VMEM((PER_W,), jnp.int32),
        pltpu.VMEM((NBUF, CHUNK, EMB), jnp.float32),
        pltpu.SemaphoreType.DMA,
        pltpu.SemaphoreType.DMA,
        pltpu.SemaphoreType.DMA,
        pltpu.SemaphoreType.DMA,
        pltpu.SemaphoreType.DMA,
        pltpu.SemaphoreType.DMA,
        pltpu.SemaphoreType.DMA,
        pltpu.SemaphoreType.DMA,
    ],
)
def _gather_kernel(src_hbm, table_hbm, out_hbm, idx_v, rows_v,
                   g0, g1, g2, g3, w0, w1, w2, w3):
    gsem = [g0, g1, g2, g3]
    wsem = [w0, w1, w2, w3]
    wid = lax.axis_index("s") * NC + lax.axis_index("c")
    base = wid * PER_W

    pltpu.sync_copy(src_hbm.at[pl.ds(base, PER_W)], idx_v)

    def fire_gather(b, g):
        pltpu.async_copy(
            table_hbm.at[idx_v.at[pl.ds(g * CHUNK, CHUNK)]],
            rows_v.at[b], gsem[b])

    def fire_wb(b, g):
        pltpu.async_copy(
            rows_v.at[b], out_hbm.at[pl.ds(base + g * CHUNK, CHUNK)],
            wsem[b])

    def drain(sem, b):
        pltpu.make_async_copy(
            out_hbm.at[pl.ds(base, CHUNK)], rows_v.at[b], sem).wait()

    fire_gather(0, 0)
    fire_gather(1, 1)

    def cycle(i, carry):
        for b in range(NBUF):
            g = i * NBUF + b
            drain(gsem[b], b)
            fire_wb(b, g)
            rb = (b + 2) % NBUF
            rg = g + 2

            @pl.when(rg < STEPS)
            def _():
                if b < 2:
                    @pl.when(i > 0)
                    def _():
                        drain(wsem[rb], rb)
                else:
                    drain(wsem[rb], rb)
                fire_gather(rb, rg)

        return carry

    lax.fori_loop(0, OUTER, cycle, 0)

    for b in range(NBUF):
        drain(wsem[b], b)


def kernel(src, mask, emb_table):
    del mask
    out = _gather_kernel(src.reshape(N), emb_table)
    return out.reshape(B, L, EMB)
